# baseline (speedup 1.0000x reference)
import functools

import jax
import jax.numpy as jnp
from jax import lax
from jax.experimental import pallas as pl
from jax.experimental.pallas import tpu as pltpu
from jax.experimental.pallas import tpu_sc as plsc

N = 10000
E = 320000
D = 128
G = 10
EPS = 1e-5

NCORE = 2
NSUB = 16
NTILE = NCORE * NSUB

CHP = 128
CPT = 80
ETP = CPT * CHP
EP = NTILE * ETP
NBUF = 3
RPT = 632
RPT_LAST = N - (NSUB - 1) * RPT
ZR = 8
NZIT = RPT // ZR
NZIT_LAST = RPT_LAST // ZR

DW = 16

R = 400
NB = N // R

_f32 = jnp.float32


def _mesh():
    return plsc.VectorSubcoreMesh(core_axis_name="c", subcore_axis_name="s",
                                  num_cores=NCORE, num_subcores=NSUB)


def _sc_degrees(srcp, dstp):
    @functools.partial(
        pl.kernel,
        out_type=(jax.ShapeDtypeStruct((NCORE * N, D), _f32),
                  jax.ShapeDtypeStruct((NCORE * N, D), _f32)),
        mesh=_mesh(),
        scratch_types=[
            pltpu.VMEM((CHP, D), _f32),
            pltpu.VMEM((ZR, D), _f32),
            pltpu.VMEM((CHP,), jnp.int32),
            pltpu.VMEM_SHARED((N + 8, D), _f32),
        ],
    )
    def deg_kernel(src_hbm, dst_hbm, degs_out, degd_out,
                   ones_v, zb, idxv, acc):
        c = lax.axis_index("c")
        s = lax.axis_index("s")
        g = c * NSUB + s
        zero16 = jnp.zeros((16,), _f32)
        one16 = jnp.ones((16,), _f32)
        for r in range(ZR):
            for j in range(D // 16):
                zb[r, pl.ds(j * 16, 16)] = zero16
        for r in range(CHP):
            for j in range(D // 16):
                ones_v[r, pl.ds(j * 16, 16)] = one16
        row0 = s * RPT
        nzit = jnp.where(s == NSUB - 1, NZIT_LAST, NZIT)
        base = g * ETP

        def zero_acc():
            def zbody(i, carry):
                pltpu.sync_copy(zb, acc.at[pl.ds(row0 + i * ZR, ZR)])
                return carry
            lax.fori_loop(0, nzit, zbody, None)

        def hist(edge_hbm):
            def ebody(i, carry):
                off = base + i * CHP
                pltpu.sync_copy(edge_hbm.at[pl.ds(off, CHP)], idxv)
                pltpu.sync_copy(ones_v, acc.at[idxv], add=True)
                return carry
            lax.fori_loop(0, CPT, ebody, None)

        def copy_out(out_hbm):
            @pl.when(s < NSUB - 1)
            def _():
                pltpu.sync_copy(acc.at[pl.ds(row0, RPT)],
                                out_hbm.at[pl.ds(c * N + row0, RPT)])

            @pl.when(s == NSUB - 1)
            def _():
                pltpu.sync_copy(acc.at[pl.ds(row0, RPT_LAST)],
                                out_hbm.at[pl.ds(c * N + row0, RPT_LAST)])

        zero_acc()
        plsc.subcore_barrier()
        hist(src_hbm)
        plsc.subcore_barrier()
        copy_out(degs_out)
        zero_acc()
        plsc.subcore_barrier()
        hist(dst_hbm)
        plsc.subcore_barrier()
        copy_out(degd_out)

    return deg_kernel(srcp, dstp)


def _sc_spmm(hmat, srcm, dstm):
    @functools.partial(
        pl.kernel,
        out_type=jax.ShapeDtypeStruct((NCORE * N, D), _f32),
        mesh=_mesh(),
        scratch_types=[
            [pltpu.VMEM((CHP,), jnp.int32) for _ in range(NBUF)],
            [pltpu.VMEM((CHP,), jnp.int32) for _ in range(NBUF)],
            [pltpu.VMEM((CHP, D), _f32) for _ in range(NBUF)],
            pltpu.VMEM_SHARED((N + 8, D), _f32),
            [pltpu.SemaphoreType.DMA for _ in range(NBUF)],
            [pltpu.SemaphoreType.DMA for _ in range(NBUF)],
        ],
    )
    def spmm_kernel(h_hbm, src_hbm, dst_hbm, out_hbm,
                    sidx, didx, rows, acc, gsem, ssem):
        c = lax.axis_index("c")
        s = lax.axis_index("s")
        g = c * NSUB + s
        zero16 = jnp.zeros((16,), _f32)
        for r in range(ZR):
            for j in range(D // 16):
                rows[0][r, pl.ds(j * 16, 16)] = zero16
        row0 = s * RPT
        nzit = jnp.where(s == NSUB - 1, NZIT_LAST, NZIT)

        def zbody(i, carry):
            pltpu.sync_copy(rows[0].at[pl.ds(0, ZR)],
                            acc.at[pl.ds(row0 + i * ZR, ZR)])
            return carry
        lax.fori_loop(0, nzit, zbody, None)
        plsc.subcore_barrier()

        base = g * ETP

        def load_idx(b, i):
            pltpu.sync_copy(src_hbm.at[pl.ds(base + i * CHP, CHP)], sidx[b])
            pltpu.sync_copy(dst_hbm.at[pl.ds(base + i * CHP, CHP)], didx[b])

        def issue_gather(b):
            pltpu.async_copy(h_hbm.at[sidx[b]], rows[b], gsem[b])

        def wait_gather(b):
            pltpu.make_async_copy(h_hbm.at[sidx[b]], rows[b],
                                  gsem[b]).wait()

        def issue_scatter(b):
            pltpu.async_copy(rows[b], acc.at[didx[b]], ssem[b], add=True)

        def wait_scatter(b):
            pltpu.make_async_copy(rows[b], acc.at[didx[b]],
                                  ssem[b]).wait()

        load_idx(0, 0)
        issue_gather(0)
        load_idx(1, 1)
        issue_gather(1)

        def turn(i, b, prefetch):
            wait_gather(b)
            issue_scatter(b)
            nb = (b + 2) % NBUF

            @pl.when(i >= 1)
            def _():
                wait_scatter(nb)
            if prefetch:
                load_idx(nb, i + 2)
                issue_gather(nb)

        def kbody(k, carry):
            for b in range(NBUF):
                turn(k * NBUF + b, b, True)
            return carry
        lax.fori_loop(0, (CPT - 2) // NBUF, kbody, None)
        turn(CPT - 2, (CPT - 2) % NBUF, False)
        turn(CPT - 1, (CPT - 1) % NBUF, False)
        wait_scatter((CPT - 1) % NBUF)
        plsc.subcore_barrier()

        @pl.when(s < NSUB - 1)
        def _():
            pltpu.sync_copy(acc.at[pl.ds(row0, RPT)],
                            out_hbm.at[pl.ds(c * N + row0, RPT)])

        @pl.when(s == NSUB - 1)
        def _():
            pltpu.sync_copy(acc.at[pl.ds(row0, RPT_LAST)],
                            out_hbm.at[pl.ds(c * N + row0, RPT_LAST)])

    return spmm_kernel(hmat, srcm, dstm)


def _dot(a, b):
    return lax.dot_general(a, b, (((1,), (0,)), ((), ())),
                           preferred_element_type=_f32)


def _bcast(col):
    return jnp.broadcast_to(col[:, 0:1], (R, D))


def _tc_prep(x, degs, degd):
    def body(x_ref, ds_ref, dd_ref, xs_ref, ii_ref, io_ref):
        inv_o = lax.rsqrt(jnp.maximum(ds_ref[0] + ds_ref[1], 1.0))
        inv_i = lax.rsqrt(jnp.maximum(dd_ref[0] + dd_ref[1], 1.0))
        xs_ref[...] = x_ref[...] * inv_o
        ii_ref[...] = inv_i[:, :DW]
        io_ref[...] = inv_o[:, :DW]

    return pl.pallas_call(
        body,
        grid=(NB,),
        in_specs=[
            pl.BlockSpec((R, D), lambda j: (j, 0)),
            pl.BlockSpec((NCORE, R, D), lambda j: (0, j, 0)),
            pl.BlockSpec((NCORE, R, D), lambda j: (0, j, 0)),
        ],
        out_specs=[
            pl.BlockSpec((R, D), lambda j: (j, 0)),
            pl.BlockSpec((R, DW), lambda j: (j, 0)),
            pl.BlockSpec((R, DW), lambda j: (j, 0)),
        ],
        out_shape=[
            jax.ShapeDtypeStruct((N + 8, D), _f32),
            jax.ShapeDtypeStruct((N, DW), _f32),
            jax.ShapeDtypeStruct((N, DW), _f32),
        ],
    )(x, degs, degd)


def _tc_post1(aggp, ii16, io16, W, b, gamma, beta):
    def body(agg_ref, ii_ref, io_ref, w_ref, b_ref, g_ref, be_ref,
             out_ref, acc_ref):
        ph = pl.program_id(0)
        j = pl.program_id(1)
        a = (agg_ref[0] + agg_ref[1]) * _bcast(ii_ref[...])
        p = _dot(a, w_ref[...]) + b_ref[...]

        @pl.when((ph == 0) & (j == 0))
        def _():
            acc_ref[...] = jnp.zeros_like(acc_ref)

        @pl.when(ph == 0)
        def _():
            acc_ref[0:1] = acc_ref[0:1] + jnp.sum(p, axis=0, keepdims=True)
            acc_ref[1:2] = acc_ref[1:2] + jnp.sum(p * p, axis=0, keepdims=True)

        @pl.when(ph == 1)
        def _():
            mu = acc_ref[0:1] / N
            var = acc_ref[1:2] / N - mu * mu
            rstd = lax.rsqrt(var + EPS)
            h = jnp.maximum((p - mu) * rstd * g_ref[...] + be_ref[...], 0.0)
            out_ref[...] = h * _bcast(io_ref[...])

    return pl.pallas_call(
        body,
        grid=(2, NB),
        in_specs=[
            pl.BlockSpec((NCORE, R, D), lambda p, j: (0, j, 0)),
            pl.BlockSpec((R, DW), lambda p, j: (j, 0)),
            pl.BlockSpec((R, DW), lambda p, j: (j, 0)),
            pl.BlockSpec((D, D), lambda p, j: (0, 0)),
            pl.BlockSpec((1, D), lambda p, j: (0, 0)),
            pl.BlockSpec((1, D), lambda p, j: (0, 0)),
            pl.BlockSpec((1, D), lambda p, j: (0, 0)),
        ],
        out_specs=pl.BlockSpec((R, D), lambda p, j: (j, 0)),
        out_shape=jax.ShapeDtypeStruct((N + 8, D), _f32),
        scratch_shapes=[pltpu.VMEM((8, D), _f32)],
    )(aggp, ii16, io16, W, b, gamma, beta)


def _tc_post2(aggp, ii16, batch_r, W, b, gamma, beta, ggv, bgv,
              wr_p, br_p, wc_p, bc_p):
    def body(agg_ref, ii_ref, bt_ref, w_ref, b_ref, g_ref, be_ref,
             gg_ref, bg_ref, wr_ref, br_ref, wc_ref, bc_ref,
             h_ref, y_ref, cc_ref, acc_ref, gsum_ref, gcnt_ref):
        ph = pl.program_id(0)
        j = pl.program_id(1)
        a = (agg_ref[0] + agg_ref[1]) * _bcast(ii_ref[...])
        p = _dot(a, w_ref[...]) + b_ref[...]

        @pl.when((ph == 0) & (j == 0))
        def _():
            acc_ref[...] = jnp.zeros_like(acc_ref)
            gsum_ref[...] = jnp.zeros_like(gsum_ref)
            gcnt_ref[...] = jnp.zeros_like(gcnt_ref)

        @pl.when(ph == 0)
        def _():
            acc_ref[0:1] = acc_ref[0:1] + jnp.sum(p, axis=0, keepdims=True)
            acc_ref[1:2] = acc_ref[1:2] + jnp.sum(p * p, axis=0, keepdims=True)

        @pl.when(ph == 1)
        def _():
            mu = acc_ref[0:1] / N
            var = acc_ref[1:2] / N - mu * mu
            rstd = lax.rsqrt(var + EPS)
            h = jnp.maximum((p - mu) * rstd * g_ref[...] + be_ref[...], 0.0)
            h_ref[...] = h
            bt = bt_ref[0]
            gi = lax.broadcasted_iota(jnp.int32, (16, R), 0)
            oh = (gi == jnp.broadcast_to(bt, (16, R))).astype(_f32)
            gsum_ref[...] = gsum_ref[...] + _dot(oh, h)
            gcnt_ref[...] = gcnt_ref[...] + jnp.broadcast_to(
                jnp.sum(oh, axis=1, keepdims=True), (16, D))

        @pl.when((ph == 1) & (j == NB - 1))
        def _():
            cnt = jnp.maximum(gcnt_ref[...], 1.0)
            gemb = gsum_ref[...] / cnt
            rmask = (lax.broadcasted_iota(jnp.int32, (16, D), 0) < G)
            rmaskf = rmask.astype(_f32)
            gm = jnp.sum(gemb * rmaskf, axis=0, keepdims=True) / G
            gv = jnp.sum(((gemb - gm) ** 2) * rmaskf, axis=0,
                         keepdims=True) / G
            gn = (gemb - gm) * lax.rsqrt(gv + EPS) * gg_ref[...] + bg_ref[...]
            y_ref[...] = _dot(gn, wr_ref[...]) + br_ref[...]
            cc_ref[...] = _dot(gn, wc_ref[...]) + bc_ref[...]

    return pl.pallas_call(
        body,
        grid=(2, NB),
        in_specs=[
            pl.BlockSpec((NCORE, R, D), lambda p, j: (0, j, 0)),
            pl.BlockSpec((R, DW), lambda p, j: (j, 0)),
            pl.BlockSpec((1, 1, R), lambda p, j: (j, 0, 0)),
            pl.BlockSpec((D, D), lambda p, j: (0, 0)),
            pl.BlockSpec((1, D), lambda p, j: (0, 0)),
            pl.BlockSpec((1, D), lambda p, j: (0, 0)),
            pl.BlockSpec((1, D), lambda p, j: (0, 0)),
            pl.BlockSpec((1, D), lambda p, j: (0, 0)),
            pl.BlockSpec((1, D), lambda p, j: (0, 0)),
            pl.BlockSpec((D, D), lambda p, j: (0, 0)),
            pl.BlockSpec((1, D), lambda p, j: (0, 0)),
            pl.BlockSpec((D, D), lambda p, j: (0, 0)),
            pl.BlockSpec((1, D), lambda p, j: (0, 0)),
        ],
        out_specs=[
            pl.BlockSpec((R, D), lambda p, j: (j, 0)),
            pl.BlockSpec((16, D), lambda p, j: (0, 0)),
            pl.BlockSpec((16, D), lambda p, j: (0, 0)),
        ],
        out_shape=[
            jax.ShapeDtypeStruct((N, D), _f32),
            jax.ShapeDtypeStruct((16, D), _f32),
            jax.ShapeDtypeStruct((16, D), _f32),
        ],
        scratch_shapes=[pltpu.VMEM((8, D), _f32),
                        pltpu.VMEM((16, D), _f32),
                        pltpu.VMEM((16, D), _f32)],
    )(aggp, ii16, batch_r, W, b, gamma, beta, ggv, bgv,
      wr_p, br_p, wc_p, bc_p)


def kernel(x, edge_index, batch, W1, b1, g1, be1, W2, b2, g2, be2,
           gg, bg, Wr, br, Wc, bc):
    src = edge_index[0]
    dst = edge_index[1]

    npad = EP - E
    pad = jnp.full((npad,), N, jnp.int32)
    src_p = jnp.concatenate([src, pad])
    dst_p = jnp.concatenate([dst, pad])

    degs2, degd2 = _sc_degrees(src_p, dst_p)
    degs = degs2.reshape(NCORE, N, D)
    degd = degd2.reshape(NCORE, N, D)

    xs, ii16, io16 = _tc_prep(x, degs, degd)

    agg1 = _sc_spmm(xs, src_p, dst_p).reshape(NCORE, N, D)
    h1s = _tc_post1(agg1, ii16, io16, W1,
                    b1.reshape(1, D), g1.reshape(1, D), be1.reshape(1, D))

    agg2 = _sc_spmm(h1s, src_p, dst_p).reshape(NCORE, N, D)

    nout = Wr.shape[1]
    ncpt = Wc.shape[1]
    wr_p = jnp.pad(Wr, ((0, 0), (0, D - nout)))
    br_p = jnp.pad(br, (0, D - nout)).reshape(1, D)
    wc_p = jnp.pad(Wc, ((0, 0), (0, D - ncpt)))
    bc_p = jnp.pad(bc, (0, D - ncpt)).reshape(1, D)
    batch_r = batch.reshape(NB, 1, R)

    h, y_f, c_f = _tc_post2(agg2, ii16, batch_r, W2,
                            b2.reshape(1, D), g2.reshape(1, D),
                            be2.reshape(1, D), gg.reshape(1, D),
                            bg.reshape(1, D), wr_p, br_p, wc_p, bc_p)
    y = y_f[:G, :nout]
    concept = c_f[:G, :ncpt]
    return (h, y, concept)


# speedup vs baseline: 1.0005x; 1.0005x over previous
import functools

import jax
import jax.numpy as jnp
from jax import lax
from jax.experimental import pallas as pl
from jax.experimental.pallas import tpu as pltpu
from jax.experimental.pallas import tpu_sc as plsc

N = 10000
E = 320000
D = 128
G = 10
EPS = 1e-5

NCORE = 2
NSUB = 16
NTILE = NCORE * NSUB

CHP = 128
CPT = 80
ETP = CPT * CHP
EP = NTILE * ETP
NBUF = 3
ISL = 2 * NBUF
RPT = 632
RPT_LAST = N - (NSUB - 1) * RPT
ZR = 8
NZIT = RPT // ZR
NZIT_LAST = RPT_LAST // ZR

DW = 16

R = 400
NB = N // R

_f32 = jnp.float32


def _mesh():
    return plsc.VectorSubcoreMesh(core_axis_name="c", subcore_axis_name="s",
                                  num_cores=NCORE, num_subcores=NSUB)


def _sc_degrees(srcp, dstp):
    @functools.partial(
        pl.kernel,
        out_type=(jax.ShapeDtypeStruct((NCORE * N, D), _f32),
                  jax.ShapeDtypeStruct((NCORE * N, D), _f32)),
        mesh=_mesh(),
        scratch_types=[
            pltpu.VMEM((CHP, D), _f32),
            pltpu.VMEM((ZR, D), _f32),
            pltpu.VMEM((CHP,), jnp.int32),
            pltpu.VMEM_SHARED((N + 8, D), _f32),
        ],
    )
    def deg_kernel(src_hbm, dst_hbm, degs_out, degd_out,
                   ones_v, zb, idxv, acc):
        c = lax.axis_index("c")
        s = lax.axis_index("s")
        g = c * NSUB + s
        zero16 = jnp.zeros((16,), _f32)
        one16 = jnp.ones((16,), _f32)
        for r in range(ZR):
            for j in range(D // 16):
                zb[r, pl.ds(j * 16, 16)] = zero16
        for r in range(CHP):
            for j in range(D // 16):
                ones_v[r, pl.ds(j * 16, 16)] = one16
        row0 = s * RPT
        nzit = jnp.where(s == NSUB - 1, NZIT_LAST, NZIT)
        base = g * ETP

        def zero_acc():
            def zbody(i, carry):
                pltpu.sync_copy(zb, acc.at[pl.ds(row0 + i * ZR, ZR)])
                return carry
            lax.fori_loop(0, nzit, zbody, None)

        def hist(edge_hbm):
            def ebody(i, carry):
                off = base + i * CHP
                pltpu.sync_copy(edge_hbm.at[pl.ds(off, CHP)], idxv)
                pltpu.sync_copy(ones_v, acc.at[idxv], add=True)
                return carry
            lax.fori_loop(0, CPT, ebody, None)

        def copy_out(out_hbm):
            @pl.when(s < NSUB - 1)
            def _():
                pltpu.sync_copy(acc.at[pl.ds(row0, RPT)],
                                out_hbm.at[pl.ds(c * N + row0, RPT)])

            @pl.when(s == NSUB - 1)
            def _():
                pltpu.sync_copy(acc.at[pl.ds(row0, RPT_LAST)],
                                out_hbm.at[pl.ds(c * N + row0, RPT_LAST)])

        zero_acc()
        plsc.subcore_barrier()
        hist(src_hbm)
        plsc.subcore_barrier()
        copy_out(degs_out)
        zero_acc()
        plsc.subcore_barrier()
        hist(dst_hbm)
        plsc.subcore_barrier()
        copy_out(degd_out)

    return deg_kernel(srcp, dstp)


def _sc_spmm(hmat, srcm, dstm):
    @functools.partial(
        pl.kernel,
        out_type=jax.ShapeDtypeStruct((NCORE * N, D), _f32),
        mesh=_mesh(),
        scratch_types=[
            [pltpu.VMEM((CHP,), jnp.int32) for _ in range(ISL)],
            [pltpu.VMEM((CHP,), jnp.int32) for _ in range(ISL)],
            [pltpu.VMEM((CHP, D), _f32) for _ in range(NBUF)],
            pltpu.VMEM_SHARED((N + 8, D), _f32),
            [pltpu.SemaphoreType.DMA for _ in range(NBUF)],
            [pltpu.SemaphoreType.DMA for _ in range(NBUF)],
            [pltpu.SemaphoreType.DMA for _ in range(ISL)],
            [pltpu.SemaphoreType.DMA for _ in range(ISL)],
        ],
    )
    def spmm_kernel(h_hbm, src_hbm, dst_hbm, out_hbm,
                    sidx, didx, rows, acc, gsem, ssem, sisem, disem):
        c = lax.axis_index("c")
        s = lax.axis_index("s")
        g = c * NSUB + s
        zero16 = jnp.zeros((16,), _f32)
        for r in range(ZR):
            for j in range(D // 16):
                rows[0][r, pl.ds(j * 16, 16)] = zero16
        row0 = s * RPT
        nzit = jnp.where(s == NSUB - 1, NZIT_LAST, NZIT)

        def zbody(i, carry):
            pltpu.sync_copy(rows[0].at[pl.ds(0, ZR)],
                            acc.at[pl.ds(row0 + i * ZR, ZR)])
            return carry
        lax.fori_loop(0, nzit, zbody, None)
        plsc.subcore_barrier()

        base = g * ETP

        def load_idx_sync(q, i):
            pltpu.sync_copy(src_hbm.at[pl.ds(base + i * CHP, CHP)], sidx[q])
            pltpu.sync_copy(dst_hbm.at[pl.ds(base + i * CHP, CHP)], didx[q])

        def issue_idx(q, i):
            pltpu.async_copy(src_hbm.at[pl.ds(base + i * CHP, CHP)],
                             sidx[q], sisem[q])
            pltpu.async_copy(dst_hbm.at[pl.ds(base + i * CHP, CHP)],
                             didx[q], disem[q])

        def wait_idx(q, i):
            pltpu.make_async_copy(src_hbm.at[pl.ds(base + i * CHP, CHP)],
                                  sidx[q], sisem[q]).wait()
            pltpu.make_async_copy(dst_hbm.at[pl.ds(base + i * CHP, CHP)],
                                  didx[q], disem[q]).wait()

        def issue_gather(b, q):
            pltpu.async_copy(h_hbm.at[sidx[q]], rows[b], gsem[b])

        def wait_gather(b, q):
            pltpu.make_async_copy(h_hbm.at[sidx[q]], rows[b],
                                  gsem[b]).wait()

        def issue_scatter(b, q):
            pltpu.async_copy(rows[b], acc.at[didx[q]], ssem[b], add=True)

        def wait_scatter(b, q):
            pltpu.make_async_copy(rows[b], acc.at[didx[q]],
                                  ssem[b]).wait()

        PF = NBUF - 1
        for k in range(PF):
            load_idx_sync(k, k)
            issue_gather(k, k)
        for k in range(PF, min(PF + 2, CPT)):
            issue_idx(k % ISL, k)

        def turn(i, b, q, prefetch):
            wait_gather(b, q)
            issue_scatter(b, q)
            nb = (b + PF) % NBUF
            nq = (q + PF) % ISL

            @pl.when(i >= 1)
            def _():
                wait_scatter(nb, (q - 1) % ISL)
            if prefetch:
                wait_idx(nq, i + PF)
                issue_gather(nb, nq)

                @pl.when(i + PF + 2 <= CPT - 1)
                def _():
                    issue_idx((q + PF + 2) % ISL, i + PF + 2)

        NPF = CPT - PF
        KB_ = NPF // ISL
        REM = NPF % ISL

        def kbody(k, carry):
            i0 = k * ISL
            for t in range(ISL):
                turn(i0 + t, t % NBUF, t, True)
            return carry
        lax.fori_loop(0, KB_, kbody, None)
        for j in range(REM):
            i = KB_ * ISL + j
            turn(i, i % NBUF, i % ISL, True)
        for j in range(PF):
            i = NPF + j
            turn(i, i % NBUF, i % ISL, False)
        wait_scatter((CPT - 1) % NBUF, (CPT - 1) % ISL)
        plsc.subcore_barrier()

        @pl.when(s < NSUB - 1)
        def _():
            pltpu.sync_copy(acc.at[pl.ds(row0, RPT)],
                            out_hbm.at[pl.ds(c * N + row0, RPT)])

        @pl.when(s == NSUB - 1)
        def _():
            pltpu.sync_copy(acc.at[pl.ds(row0, RPT_LAST)],
                            out_hbm.at[pl.ds(c * N + row0, RPT_LAST)])

    return spmm_kernel(hmat, srcm, dstm)


def _dot(a, b):
    return lax.dot_general(a, b, (((1,), (0,)), ((), ())),
                           preferred_element_type=_f32)


def _bcast(col):
    return jnp.broadcast_to(col[:, 0:1], (R, D))


def _tc_prep(x, degs, degd):
    def body(x_ref, ds_ref, dd_ref, xs_ref, ii_ref, io_ref):
        inv_o = lax.rsqrt(jnp.maximum(ds_ref[0] + ds_ref[1], 1.0))
        inv_i = lax.rsqrt(jnp.maximum(dd_ref[0] + dd_ref[1], 1.0))
        xs_ref[...] = x_ref[...] * inv_o
        ii_ref[...] = inv_i[:, :DW]
        io_ref[...] = inv_o[:, :DW]

    return pl.pallas_call(
        body,
        grid=(NB,),
        in_specs=[
            pl.BlockSpec((R, D), lambda j: (j, 0)),
            pl.BlockSpec((NCORE, R, D), lambda j: (0, j, 0)),
            pl.BlockSpec((NCORE, R, D), lambda j: (0, j, 0)),
        ],
        out_specs=[
            pl.BlockSpec((R, D), lambda j: (j, 0)),
            pl.BlockSpec((R, DW), lambda j: (j, 0)),
            pl.BlockSpec((R, DW), lambda j: (j, 0)),
        ],
        out_shape=[
            jax.ShapeDtypeStruct((N + 8, D), _f32),
            jax.ShapeDtypeStruct((N, DW), _f32),
            jax.ShapeDtypeStruct((N, DW), _f32),
        ],
    )(x, degs, degd)


def _tc_post1(aggp, ii16, io16, W, b, gamma, beta):
    def body(agg_ref, ii_ref, io_ref, w_ref, b_ref, g_ref, be_ref,
             out_ref, acc_ref):
        ph = pl.program_id(0)
        j = pl.program_id(1)
        a = (agg_ref[0] + agg_ref[1]) * _bcast(ii_ref[...])
        p = _dot(a, w_ref[...]) + b_ref[...]

        @pl.when((ph == 0) & (j == 0))
        def _():
            acc_ref[...] = jnp.zeros_like(acc_ref)

        @pl.when(ph == 0)
        def _():
            acc_ref[0:1] = acc_ref[0:1] + jnp.sum(p, axis=0, keepdims=True)
            acc_ref[1:2] = acc_ref[1:2] + jnp.sum(p * p, axis=0, keepdims=True)

        @pl.when(ph == 1)
        def _():
            mu = acc_ref[0:1] / N
            var = acc_ref[1:2] / N - mu * mu
            rstd = lax.rsqrt(var + EPS)
            h = jnp.maximum((p - mu) * rstd * g_ref[...] + be_ref[...], 0.0)
            out_ref[...] = h * _bcast(io_ref[...])

    return pl.pallas_call(
        body,
        grid=(2, NB),
        in_specs=[
            pl.BlockSpec((NCORE, R, D), lambda p, j: (0, j, 0)),
            pl.BlockSpec((R, DW), lambda p, j: (j, 0)),
            pl.BlockSpec((R, DW), lambda p, j: (j, 0)),
            pl.BlockSpec((D, D), lambda p, j: (0, 0)),
            pl.BlockSpec((1, D), lambda p, j: (0, 0)),
            pl.BlockSpec((1, D), lambda p, j: (0, 0)),
            pl.BlockSpec((1, D), lambda p, j: (0, 0)),
        ],
        out_specs=pl.BlockSpec((R, D), lambda p, j: (j, 0)),
        out_shape=jax.ShapeDtypeStruct((N + 8, D), _f32),
        scratch_shapes=[pltpu.VMEM((8, D), _f32)],
    )(aggp, ii16, io16, W, b, gamma, beta)


def _tc_post2(aggp, ii16, batch_r, W, b, gamma, beta, ggv, bgv,
              wr_p, br_p, wc_p, bc_p):
    def body(agg_ref, ii_ref, bt_ref, w_ref, b_ref, g_ref, be_ref,
             gg_ref, bg_ref, wr_ref, br_ref, wc_ref, bc_ref,
             h_ref, y_ref, cc_ref, acc_ref, gsum_ref, gcnt_ref):
        ph = pl.program_id(0)
        j = pl.program_id(1)
        a = (agg_ref[0] + agg_ref[1]) * _bcast(ii_ref[...])
        p = _dot(a, w_ref[...]) + b_ref[...]

        @pl.when((ph == 0) & (j == 0))
        def _():
            acc_ref[...] = jnp.zeros_like(acc_ref)
            gsum_ref[...] = jnp.zeros_like(gsum_ref)
            gcnt_ref[...] = jnp.zeros_like(gcnt_ref)

        @pl.when(ph == 0)
        def _():
            acc_ref[0:1] = acc_ref[0:1] + jnp.sum(p, axis=0, keepdims=True)
            acc_ref[1:2] = acc_ref[1:2] + jnp.sum(p * p, axis=0, keepdims=True)

        @pl.when(ph == 1)
        def _():
            mu = acc_ref[0:1] / N
            var = acc_ref[1:2] / N - mu * mu
            rstd = lax.rsqrt(var + EPS)
            h = jnp.maximum((p - mu) * rstd * g_ref[...] + be_ref[...], 0.0)
            h_ref[...] = h
            bt = bt_ref[0]
            gi = lax.broadcasted_iota(jnp.int32, (16, R), 0)
            oh = (gi == jnp.broadcast_to(bt, (16, R))).astype(_f32)
            gsum_ref[...] = gsum_ref[...] + _dot(oh, h)
            gcnt_ref[...] = gcnt_ref[...] + jnp.broadcast_to(
                jnp.sum(oh, axis=1, keepdims=True), (16, D))

        @pl.when((ph == 1) & (j == NB - 1))
        def _():
            cnt = jnp.maximum(gcnt_ref[...], 1.0)
            gemb = gsum_ref[...] / cnt
            rmask = (lax.broadcasted_iota(jnp.int32, (16, D), 0) < G)
            rmaskf = rmask.astype(_f32)
            gm = jnp.sum(gemb * rmaskf, axis=0, keepdims=True) / G
            gv = jnp.sum(((gemb - gm) ** 2) * rmaskf, axis=0,
                         keepdims=True) / G
            gn = (gemb - gm) * lax.rsqrt(gv + EPS) * gg_ref[...] + bg_ref[...]
            y_ref[...] = _dot(gn, wr_ref[...]) + br_ref[...]
            cc_ref[...] = _dot(gn, wc_ref[...]) + bc_ref[...]

    return pl.pallas_call(
        body,
        grid=(2, NB),
        in_specs=[
            pl.BlockSpec((NCORE, R, D), lambda p, j: (0, j, 0)),
            pl.BlockSpec((R, DW), lambda p, j: (j, 0)),
            pl.BlockSpec((1, 1, R), lambda p, j: (j, 0, 0)),
            pl.BlockSpec((D, D), lambda p, j: (0, 0)),
            pl.BlockSpec((1, D), lambda p, j: (0, 0)),
            pl.BlockSpec((1, D), lambda p, j: (0, 0)),
            pl.BlockSpec((1, D), lambda p, j: (0, 0)),
            pl.BlockSpec((1, D), lambda p, j: (0, 0)),
            pl.BlockSpec((1, D), lambda p, j: (0, 0)),
            pl.BlockSpec((D, D), lambda p, j: (0, 0)),
            pl.BlockSpec((1, D), lambda p, j: (0, 0)),
            pl.BlockSpec((D, D), lambda p, j: (0, 0)),
            pl.BlockSpec((1, D), lambda p, j: (0, 0)),
        ],
        out_specs=[
            pl.BlockSpec((R, D), lambda p, j: (j, 0)),
            pl.BlockSpec((16, D), lambda p, j: (0, 0)),
            pl.BlockSpec((16, D), lambda p, j: (0, 0)),
        ],
        out_shape=[
            jax.ShapeDtypeStruct((N, D), _f32),
            jax.ShapeDtypeStruct((16, D), _f32),
            jax.ShapeDtypeStruct((16, D), _f32),
        ],
        scratch_shapes=[pltpu.VMEM((8, D), _f32),
                        pltpu.VMEM((16, D), _f32),
                        pltpu.VMEM((16, D), _f32)],
    )(aggp, ii16, batch_r, W, b, gamma, beta, ggv, bgv,
      wr_p, br_p, wc_p, bc_p)


def kernel(x, edge_index, batch, W1, b1, g1, be1, W2, b2, g2, be2,
           gg, bg, Wr, br, Wc, bc):
    src = edge_index[0]
    dst = edge_index[1]

    npad = EP - E
    pad = jnp.full((npad,), N, jnp.int32)
    src_p = jnp.concatenate([src, pad])
    dst_p = jnp.concatenate([dst, pad])

    degs2, degd2 = _sc_degrees(src_p, dst_p)
    degs = degs2.reshape(NCORE, N, D)
    degd = degd2.reshape(NCORE, N, D)

    xs, ii16, io16 = _tc_prep(x, degs, degd)

    agg1 = _sc_spmm(xs, src_p, dst_p).reshape(NCORE, N, D)
    h1s = _tc_post1(agg1, ii16, io16, W1,
                    b1.reshape(1, D), g1.reshape(1, D), be1.reshape(1, D))

    agg2 = _sc_spmm(h1s, src_p, dst_p).reshape(NCORE, N, D)

    nout = Wr.shape[1]
    ncpt = Wc.shape[1]
    wr_p = jnp.pad(Wr, ((0, 0), (0, D - nout)))
    br_p = jnp.pad(br, (0, D - nout)).reshape(1, D)
    wc_p = jnp.pad(Wc, ((0, 0), (0, D - ncpt)))
    bc_p = jnp.pad(bc, (0, D - ncpt)).reshape(1, D)
    batch_r = batch.reshape(NB, 1, R)

    h, y_f, c_f = _tc_post2(agg2, ii16, batch_r, W2,
                            b2.reshape(1, D), g2.reshape(1, D),
                            be2.reshape(1, D), gg.reshape(1, D),
                            bg.reshape(1, D), wr_p, br_p, wc_p, bc_p)
    y = y_f[:G, :nout]
    concept = c_f[:G, :ncpt]
    return (h, y, concept)
